# Initial kernel scaffold; baseline (speedup 1.0000x reference)
import jax
import jax.numpy as jnp
from jax.experimental import pallas as pl


def kernel(activations, labels, som_vectors, cell_labels, cell_reliability):
    raise NotImplementedError("write your pallas kernel here")


# speedup vs baseline: 20.8121x; 20.8121x over previous
import jax
import jax.numpy as jnp
import numpy as np
from jax.experimental import pallas as pl
from jax.experimental.pallas import tpu as pltpu

GRID = 32
DIM = 256
CELLS = GRID * GRID
BATCH = 1024
BIG = np.float32(3.0e38)
LR0 = np.float32(0.2)
LR1 = np.float32(0.1)
LR2 = np.float32(0.05)


def _som_body(a_ref, lab_ref, pt0_ref, cl_ref, rel_ref, errt_ref, pt_ref):
    pt_ref[...] = pt0_ref[...]
    lane = jax.lax.broadcasted_iota(jnp.int32, (1, CELLS), 1)
    cx = lane // GRID
    cy = lane - cx * GRID
    cl_row = cl_ref[...]
    rel_row = rel_ref[...]

    def step(t, carry):
        a_row = a_ref[pl.ds(t, 1), :]
        a_col = jnp.swapaxes(a_row, 0, 1)
        pt = pt_ref[...]
        diff = pt - a_col
        d = jnp.sum(diff * diff, axis=0, keepdims=True)

        dmin = jnp.min(d, axis=1, keepdims=True)
        idx = jnp.min(jnp.where(d == dmin, lane, CELLS), axis=1, keepdims=True)
        bx = idx // GRID
        by = idx - bx * GRID

        lab = lab_ref[t, 0]
        dp = jnp.where(cl_row == lab, d, BIG)
        pdmin = jnp.min(dp, axis=1, keepdims=True)
        pidx = jnp.min(jnp.where(dp == pdmin, lane, CELLS), axis=1, keepdims=True)
        phot = (lane == pidx)

        relv = jnp.sum(jnp.where(phot, rel_row, 0.0), axis=1, keepdims=True) * 0.01
        proto = jnp.sum(jnp.where(phot, pt, 0.0), axis=1, keepdims=True)
        err = jnp.where(relv >= 0.95, 0.01 * relv * (proto - a_col),
                        jnp.zeros_like(a_col))
        errt_ref[pl.ds(t, 1), :] = jnp.swapaxes(err, 0, 1)

        dx = jnp.abs(cx - bx)
        dy = jnp.abs(cy - by)
        cheb = jnp.maximum(dx, dy)
        c = jnp.where(cheb == 0, LR0,
                      jnp.where(cheb == 1, LR1,
                                jnp.where(cheb == 2, LR2, 0.0))).astype(jnp.float32)
        c = jnp.where(dmin > 0.0001, c, 0.0)

        pt_ref[...] = pt - c * diff
        return carry

    jax.lax.fori_loop(0, BATCH, step, 0)


_PALLAS_KWARGS = dict(
    out_shape=jax.ShapeDtypeStruct((BATCH, DIM), jnp.float32),
    in_specs=[
        pl.BlockSpec(memory_space=pltpu.VMEM),
        pl.BlockSpec(memory_space=pltpu.SMEM),
        pl.BlockSpec(memory_space=pltpu.VMEM),
        pl.BlockSpec(memory_space=pltpu.VMEM),
        pl.BlockSpec(memory_space=pltpu.VMEM),
    ],
    out_specs=pl.BlockSpec(memory_space=pltpu.VMEM),
    scratch_shapes=[pltpu.VMEM((DIM, CELLS), jnp.float32)],
)


@jax.jit
def kernel(activations, labels, som_vectors, cell_labels, cell_reliability):
    return pl.pallas_call(_som_body, **_PALLAS_KWARGS)(
        activations,
        labels.reshape(BATCH, 1),
        som_vectors.reshape(CELLS, DIM).T,
        cell_labels.reshape(1, CELLS),
        cell_reliability.reshape(1, CELLS),
    )


# speedup vs baseline: 21.2690x; 1.0220x over previous
import jax
import jax.numpy as jnp
import numpy as np
from jax.experimental import pallas as pl
from jax.experimental.pallas import tpu as pltpu

GRID = 32
DIM = 256
BATCH = 1024
XP = 2
YP = 4
NX = GRID + 2 * XP
NY = GRID + 2 * YP
NPAD = NX * NY
C = 16
NCHUNK = BATCH // C
BIG = np.float32(3.0e38)
LR = [np.float32(0.2), np.float32(0.1), np.float32(0.05)]


def _som_body(a_ref, lab_ref, p0_ref, n0_ref, cl_ref, rel_ref, err_ref,
              p_ref, m_ref, g_ref):
    p_ref[...] = p0_ref[...]
    g_ref[...] = jax.lax.dot_general(
        a_ref[...], a_ref[...], (((1,), (1,)), ((), ())),
        preferred_element_type=jnp.float32,
        precision=jax.lax.Precision.HIGHEST)
    lane = jax.lax.broadcasted_iota(jnp.int32, (1, NPAD), 1)
    lx = lane // NY
    ly = lane - lx * NY
    lane_b = jax.lax.broadcasted_iota(jnp.int32, (1, BATCH), 1)
    sub_c = jax.lax.broadcasted_iota(jnp.int32, (C, 1), 0)
    cl_row = cl_ref[...]
    realf = jnp.where(cl_row >= 0, 1.0, 0.0).astype(jnp.float32)

    def chunk(cidx, n):
        t0 = cidx * C
        a_chunk = a_ref[pl.ds(t0, C), :]
        m_ref[...] = jax.lax.dot_general(
            a_chunk, p_ref[...], (((1,), (1,)), ((), ())),
            preferred_element_type=jnp.float32,
            precision=jax.lax.Precision.HIGHEST)

        def step(k, n):
            t = t0 + k
            m = m_ref[pl.ds(k, 1), :]
            gb = g_ref[pl.ds(t0, C), :]
            g_col = jnp.sum(jnp.where(lane_b == t, gb, 0.0),
                            axis=1, keepdims=True)
            a2 = jnp.sum(jnp.where(sub_c == k, g_col, 0.0),
                         axis=0, keepdims=True)
            d = n - 2.0 * m + a2

            dmin_s = jnp.min(d)
            idx_s = jnp.min(jnp.where(d == dmin_s, lane, NPAD))
            bx_s = idx_s // NY
            by_s = idx_s - bx_s * NY

            lab = lab_ref[0, t]
            dp = jnp.where(cl_row == lab, d, BIG)
            pdmin_s = jnp.min(dp)
            pidx_s = jnp.min(jnp.where((dp == pdmin_s) & (cl_row >= 0),
                                       lane, NPAD))

            a_row = a_ref[pl.ds(t, 1), :]
            relv = rel_ref[0, pidx_s] / 100.0
            efac = jnp.where(relv >= 0.95, 0.01 * relv, 0.0)
            proto = p_ref[pl.ds(pidx_s, 1), :]
            err_ref[pl.ds(t, 1), :] = efac * (proto - a_row)

            gate = dmin_s > 0.0001
            gatef = jnp.where(gate, 1.0, 0.0).astype(jnp.float32)

            cheb = jnp.maximum(jnp.abs(lx - bx_s), jnp.abs(ly - by_s))
            c = jnp.where(cheb == 0, LR[0],
                          jnp.where(cheb == 1, LR[1],
                                    jnp.where(cheb == 2, LR[2], 0.0)))
            c = (c * realf * gatef).astype(jnp.float32)
            omc = 1.0 - c

            m_ref[...] = m_ref[...] * omc + g_col * c
            n_new = n * omc * omc + 2.0 * c * omc * m + c * c * a2

            for dx in range(-2, 3):
                for dy in range(-2, 3):
                    r0 = idx_s + (dx * NY + dy)
                    cc = gatef * LR[max(abs(dx), abs(dy))]
                    row = p_ref[pl.ds(r0, 1), :]
                    p_ref[pl.ds(r0, 1), :] = row - cc * (row - a_row)
            return n_new

        return jax.lax.fori_loop(0, C, step, n)

    jax.lax.fori_loop(0, NCHUNK, chunk, n0_ref[...])


_PALLAS_KWARGS = dict(
    out_shape=jax.ShapeDtypeStruct((BATCH, DIM), jnp.float32),
    in_specs=[
        pl.BlockSpec(memory_space=pltpu.VMEM),
        pl.BlockSpec(memory_space=pltpu.SMEM),
        pl.BlockSpec(memory_space=pltpu.VMEM),
        pl.BlockSpec(memory_space=pltpu.VMEM),
        pl.BlockSpec(memory_space=pltpu.VMEM),
        pl.BlockSpec(memory_space=pltpu.SMEM),
    ],
    out_specs=pl.BlockSpec(memory_space=pltpu.VMEM),
    scratch_shapes=[
        pltpu.VMEM((NPAD, DIM), jnp.float32),
        pltpu.VMEM((C, NPAD), jnp.float32),
        pltpu.VMEM((BATCH, BATCH), jnp.float32),
    ],
)


@jax.jit
def kernel(activations, labels, som_vectors, cell_labels, cell_reliability):
    x = jnp.arange(GRID)
    rows = ((x[:, None] + XP) * NY + x[None, :] + YP).reshape(-1)
    pflat = som_vectors.reshape(GRID * GRID, DIM)
    p0 = jnp.zeros((NPAD, DIM), jnp.float32).at[rows].set(pflat)
    n0 = jnp.full((NPAD,), BIG, jnp.float32).at[rows].set(
        jnp.sum(pflat * pflat, axis=1)).reshape(1, NPAD)
    cl = jnp.full((NPAD,), -1, jnp.int32).at[rows].set(
        cell_labels.reshape(-1)).reshape(1, NPAD)
    rel = jnp.zeros((NPAD,), jnp.float32).at[rows].set(
        cell_reliability.reshape(-1)).reshape(1, NPAD)
    return pl.pallas_call(_som_body, **_PALLAS_KWARGS)(
        activations, labels.reshape(1, BATCH), p0, n0, cl, rel)


# speedup vs baseline: 27.1023x; 1.2743x over previous
import jax
import jax.numpy as jnp
import numpy as np
from jax.experimental import pallas as pl
from jax.experimental.pallas import tpu as pltpu

GRID = 32
DIM = 256
BATCH = 1024
XP = 2
YP = 4
NX = GRID + 2 * XP
NY = GRID + 2 * YP
NPAD = NX * NY + 16
C = 16
NCHUNK = BATCH // C
BIG = np.float32(3.0e38)
LR = [np.float32(0.2), np.float32(0.1), np.float32(0.05)]


def _som_body(a_ref, lab_ref, p0_ref, n0_ref, cl_ref, rel_ref, err_ref,
              p_ref, m_ref, g_ref):
    p_ref[...] = p0_ref[...]
    g_ref[...] = jax.lax.dot_general(
        a_ref[...], a_ref[...], (((1,), (1,)), ((), ())),
        preferred_element_type=jnp.float32,
        precision=jax.lax.Precision.HIGHEST)
    lane = jax.lax.broadcasted_iota(jnp.int32, (1, NPAD), 1)
    lx = lane // NY
    ly = lane - lx * NY
    lane_b = jax.lax.broadcasted_iota(jnp.int32, (1, BATCH), 1)
    sub_c = jax.lax.broadcasted_iota(jnp.int32, (C, 1), 0)
    sub_w = jax.lax.broadcasted_iota(jnp.int32, (176, 1), 0)
    cl_row = cl_ref[...]
    realf = jnp.where(cl_row >= 0, 1.0, 0.0).astype(jnp.float32)

    def chunk(cidx, n):
        t0 = cidx * C
        a_chunk = a_ref[pl.ds(t0, C), :]
        m_ref[...] = jax.lax.dot_general(
            a_chunk, p_ref[...], (((1,), (1,)), ((), ())),
            preferred_element_type=jnp.float32,
            precision=jax.lax.Precision.HIGHEST)

        def step(k, n):
            t = t0 + k
            m = m_ref[pl.ds(k, 1), :]
            gb = g_ref[pl.ds(t0, C), :]
            g_col = jnp.sum(jnp.where(lane_b == t, gb, 0.0),
                            axis=1, keepdims=True)
            a2 = jnp.sum(jnp.where(sub_c == k, g_col, 0.0),
                         axis=0, keepdims=True)
            d = n - 2.0 * m + a2

            dmin = jnp.min(d, axis=1, keepdims=True)
            idx_s = jnp.min(jnp.where(d == dmin, lane, NPAD))
            bx_s = idx_s // NY
            by_s = idx_s - bx_s * NY

            lab = lab_ref[0, t]
            dp = jnp.where(cl_row == lab, d, BIG)
            pdmin = jnp.min(dp, axis=1, keepdims=True)
            pidx_s = jnp.min(jnp.where((dp == pdmin) & (cl_row >= 0),
                                       lane, NPAD))

            a_row = a_ref[pl.ds(t, 1), :]
            relv = rel_ref[0, pidx_s] / 100.0
            efac = jnp.where(relv >= 0.95, 0.01 * relv, 0.0)
            proto = p_ref[pl.ds(pidx_s, 1), :]
            err_ref[pl.ds(t, 1), :] = efac * (proto - a_row)

            gatef = jnp.where(dmin > 0.0001, 1.0, 0.0).astype(jnp.float32)

            cheb = jnp.maximum(jnp.abs(lx - bx_s), jnp.abs(ly - by_s))
            c = jnp.where(cheb == 0, LR[0],
                          jnp.where(cheb == 1, LR[1],
                                    jnp.where(cheb == 2, LR[2], 0.0)))
            c = (c * realf * gatef).astype(jnp.float32)
            omc = 1.0 - c

            m_ref[...] = m_ref[...] * omc + g_col * c
            n_new = n * omc * omc + 2.0 * c * omc * m + c * c * a2

            ralign = ((idx_s - 82) // 8) * 8
            cellv = ralign + sub_w
            cxv = cellv // NY
            cyv = cellv - cxv * NY
            chw = jnp.maximum(jnp.abs(cxv - bx_s), jnp.abs(cyv - by_s))
            ccol = jnp.where(chw == 0, LR[0],
                             jnp.where(chw == 1, LR[1],
                                       jnp.where(chw == 2, LR[2], 0.0)))
            ccol = (ccol * gatef).astype(jnp.float32)
            blk = p_ref[pl.ds(ralign, 176), :]
            p_ref[pl.ds(ralign, 176), :] = blk - ccol * (blk - a_row)
            return n_new

        return jax.lax.fori_loop(0, C, step, n)

    jax.lax.fori_loop(0, NCHUNK, chunk, n0_ref[...])


_PALLAS_KWARGS = dict(
    out_shape=jax.ShapeDtypeStruct((BATCH, DIM), jnp.float32),
    in_specs=[
        pl.BlockSpec(memory_space=pltpu.VMEM),
        pl.BlockSpec(memory_space=pltpu.SMEM),
        pl.BlockSpec(memory_space=pltpu.VMEM),
        pl.BlockSpec(memory_space=pltpu.VMEM),
        pl.BlockSpec(memory_space=pltpu.VMEM),
        pl.BlockSpec(memory_space=pltpu.SMEM),
    ],
    out_specs=pl.BlockSpec(memory_space=pltpu.VMEM),
    scratch_shapes=[
        pltpu.VMEM((NPAD, DIM), jnp.float32),
        pltpu.VMEM((C, NPAD), jnp.float32),
        pltpu.VMEM((BATCH, BATCH), jnp.float32),
    ],
)


@jax.jit
def kernel(activations, labels, som_vectors, cell_labels, cell_reliability):
    x = jnp.arange(GRID)
    rows = ((x[:, None] + XP) * NY + x[None, :] + YP).reshape(-1)
    pflat = som_vectors.reshape(GRID * GRID, DIM)
    p0 = jnp.zeros((NPAD, DIM), jnp.float32).at[rows].set(pflat)
    n0 = jnp.full((NPAD,), BIG, jnp.float32).at[rows].set(
        jnp.sum(pflat * pflat, axis=1)).reshape(1, NPAD)
    cl = jnp.full((NPAD,), -1, jnp.int32).at[rows].set(
        cell_labels.reshape(-1)).reshape(1, NPAD)
    rel = jnp.zeros((NPAD,), jnp.float32).at[rows].set(
        cell_reliability.reshape(-1)).reshape(1, NPAD)
    return pl.pallas_call(_som_body, **_PALLAS_KWARGS)(
        activations, labels.reshape(1, BATCH), p0, n0, cl, rel)


# speedup vs baseline: 33.2483x; 1.2268x over previous
import jax
import jax.numpy as jnp
import numpy as np
from jax.experimental import pallas as pl
from jax.experimental.pallas import tpu as pltpu

GRID = 32
DIM = 256
BATCH = 1024
XP = 2
YP = 4
NX = GRID + 2 * XP
NY = GRID + 2 * YP
NPAD = NX * NY + 16
C = 16
NCHUNK = BATCH // C
BIG = np.float32(3.0e38)
LR = [np.float32(0.2), np.float32(0.1), np.float32(0.05)]


def _som_body(a_ref, lab_ref, p0_ref, n0_ref, cl_ref, rel_ref, err_ref,
              p_ref, m_ref, g_ref):
    p_ref[...] = p0_ref[...]
    g_ref[...] = jax.lax.dot_general(
        a_ref[...], a_ref[...], (((1,), (1,)), ((), ())),
        preferred_element_type=jnp.float32,
        precision=jax.lax.Precision.HIGHEST)
    lane = jax.lax.broadcasted_iota(jnp.int32, (1, NPAD), 1)
    lx = lane // NY
    ly = lane - lx * NY
    lane_b = jax.lax.broadcasted_iota(jnp.int32, (1, BATCH), 1)
    sub_c = jax.lax.broadcasted_iota(jnp.int32, (C, 1), 0)
    sub_w = jax.lax.broadcasted_iota(jnp.int32, (176, 1), 0)
    cl_row = cl_ref[...]
    realf = jnp.where(cl_row >= 0, 1.0, 0.0).astype(jnp.float32)

    def chunk(cidx, n):
        t0 = cidx * C
        a_chunk = a_ref[pl.ds(t0, C), :]
        m_ref[...] = jax.lax.dot_general(
            a_chunk, p_ref[...], (((1,), (1,)), ((), ())),
            preferred_element_type=jnp.float32,
            precision=jax.lax.Precision.HIGHEST)

        def step(k, n):
            t = t0 + k
            m = m_ref[pl.ds(k, 1), :]
            gb = g_ref[pl.ds(t0, C), :]
            g_col = jnp.sum(jnp.where(lane_b == t, gb, 0.0),
                            axis=1, keepdims=True)
            a2 = jnp.sum(jnp.where(sub_c == k, g_col, 0.0),
                         axis=0, keepdims=True)
            d = n - 2.0 * m + a2

            dmin = jnp.min(d, axis=1, keepdims=True)
            eqm = jnp.where(d == dmin, lane, NPAD)
            idx_v = jnp.min(eqm, axis=1, keepdims=True)
            bx_v = idx_v // NY
            by_v = idx_v - bx_v * NY
            idx_s = jnp.min(eqm)

            lab = lab_ref[0, t]
            dp = jnp.where(cl_row == lab, d, BIG)
            pdmin = jnp.min(dp, axis=1, keepdims=True)
            pidx_s = jnp.min(jnp.where((dp == pdmin) & (cl_row >= 0),
                                       lane, NPAD))

            a_row = a_ref[pl.ds(t, 1), :]
            relv = rel_ref[0, pidx_s] / 100.0
            efac = jnp.where(relv >= 0.95, 0.01 * relv, 0.0)
            proto = p_ref[pl.ds(pidx_s, 1), :]
            err_ref[pl.ds(t, 1), :] = efac * (proto - a_row)

            gatef = jnp.where(dmin > 0.0001, 1.0, 0.0).astype(jnp.float32)

            cheb = jnp.maximum(jnp.abs(lx - bx_v), jnp.abs(ly - by_v))
            c = jnp.where(cheb == 0, LR[0],
                          jnp.where(cheb == 1, LR[1],
                                    jnp.where(cheb == 2, LR[2], 0.0)))
            c = (c * realf * gatef).astype(jnp.float32)
            omc = 1.0 - c

            m_ref[...] = m_ref[...] * omc + g_col * c
            n_new = n * omc * omc + 2.0 * c * omc * m + c * c * a2

            ralign = ((idx_s - 82) // 8) * 8
            cellv = ralign + sub_w
            cxv = cellv // NY
            cyv = cellv - cxv * NY
            chw = jnp.maximum(jnp.abs(cxv - bx_v), jnp.abs(cyv - by_v))
            ccol = jnp.where(chw == 0, LR[0],
                             jnp.where(chw == 1, LR[1],
                                       jnp.where(chw == 2, LR[2], 0.0)))
            ccol = (ccol * gatef).astype(jnp.float32)
            blk = p_ref[pl.ds(ralign, 176), :]
            p_ref[pl.ds(ralign, 176), :] = blk - ccol * (blk - a_row)
            return n_new

        return jax.lax.fori_loop(0, C, step, n, unroll=True)

    jax.lax.fori_loop(0, NCHUNK, chunk, n0_ref[...])


_PALLAS_KWARGS = dict(
    out_shape=jax.ShapeDtypeStruct((BATCH, DIM), jnp.float32),
    in_specs=[
        pl.BlockSpec(memory_space=pltpu.VMEM),
        pl.BlockSpec(memory_space=pltpu.SMEM),
        pl.BlockSpec(memory_space=pltpu.VMEM),
        pl.BlockSpec(memory_space=pltpu.VMEM),
        pl.BlockSpec(memory_space=pltpu.VMEM),
        pl.BlockSpec(memory_space=pltpu.SMEM),
    ],
    out_specs=pl.BlockSpec(memory_space=pltpu.VMEM),
    scratch_shapes=[
        pltpu.VMEM((NPAD, DIM), jnp.float32),
        pltpu.VMEM((C, NPAD), jnp.float32),
        pltpu.VMEM((BATCH, BATCH), jnp.float32),
    ],
)


@jax.jit
def kernel(activations, labels, som_vectors, cell_labels, cell_reliability):
    x = jnp.arange(GRID)
    rows = ((x[:, None] + XP) * NY + x[None, :] + YP).reshape(-1)
    pflat = som_vectors.reshape(GRID * GRID, DIM)
    p0 = jnp.zeros((NPAD, DIM), jnp.float32).at[rows].set(pflat)
    n0 = jnp.full((NPAD,), BIG, jnp.float32).at[rows].set(
        jnp.sum(pflat * pflat, axis=1)).reshape(1, NPAD)
    cl = jnp.full((NPAD,), -1, jnp.int32).at[rows].set(
        cell_labels.reshape(-1)).reshape(1, NPAD)
    rel = jnp.zeros((NPAD,), jnp.float32).at[rows].set(
        cell_reliability.reshape(-1)).reshape(1, NPAD)
    return pl.pallas_call(_som_body, **_PALLAS_KWARGS)(
        activations, labels.reshape(1, BATCH), p0, n0, cl, rel)


# speedup vs baseline: 44.0486x; 1.3248x over previous
import jax
import jax.numpy as jnp
import numpy as np
from jax.experimental import pallas as pl
from jax.experimental.pallas import tpu as pltpu

GRID = 32
DIM = 256
BATCH = 1024
XP = 2
YP = 4
NX = GRID + 2 * XP
NY = GRID + 2 * YP
NPAD = NX * NY + 16
C = 16
NCHUNK = BATCH // C
BIG = np.float32(3.0e38)
LR = [np.float32(0.2), np.float32(0.1), np.float32(0.05)]


def _som_body(a_ref, lab_ref, p0_ref, n0_ref, cl_ref, rel_ref, err_ref,
              p_ref, m_ref, g_ref):
    p_ref[...] = p0_ref[...]
    g_ref[...] = jax.lax.dot_general(
        a_ref[...], a_ref[...], (((1,), (1,)), ((), ())),
        preferred_element_type=jnp.float32,
        precision=jax.lax.Precision.HIGHEST)
    lane = jax.lax.broadcasted_iota(jnp.int32, (1, NPAD), 1)
    lx = lane // NY
    ly = lane - lx * NY
    lane_b = jax.lax.broadcasted_iota(jnp.int32, (1, BATCH), 1)
    sub_c = jax.lax.broadcasted_iota(jnp.int32, (C, 1), 0)
    sub_w = jax.lax.broadcasted_iota(jnp.int32, (16, 1), 0)
    cl_row = cl_ref[...]
    realf = jnp.where(cl_row >= 0, 1.0, 0.0).astype(jnp.float32)

    def chunk(cidx, n):
        t0 = cidx * C
        a_chunk = a_ref[pl.ds(t0, C), :]
        m_ref[...] = jax.lax.dot_general(
            a_chunk, p_ref[...], (((1,), (1,)), ((), ())),
            preferred_element_type=jnp.float32,
            precision=jax.lax.Precision.HIGHEST)

        def step(k, n):
            t = t0 + k
            m = m_ref[pl.ds(k, 1), :]
            gb = g_ref[pl.ds(t0, C), :]
            g_col = jnp.sum(jnp.where(lane_b == t, gb, 0.0),
                            axis=1, keepdims=True)
            a2 = jnp.sum(jnp.where(sub_c == k, g_col, 0.0),
                         axis=0, keepdims=True)
            d = n - 2.0 * m + a2

            dmin = jnp.min(d, axis=1, keepdims=True)
            eqm = jnp.where(d == dmin, lane, NPAD)
            idx_v = jnp.min(eqm, axis=1, keepdims=True)
            bx_v = idx_v // NY
            by_v = idx_v - bx_v * NY
            idx_s = jnp.min(eqm)

            lab = lab_ref[0, t]
            dp = jnp.where(cl_row == lab, d, BIG)
            pdmin = jnp.min(dp, axis=1, keepdims=True)
            pidx_s = jnp.min(jnp.where((dp == pdmin) & (cl_row >= 0),
                                       lane, NPAD))

            a_row = a_ref[pl.ds(t, 1), :]
            relv = rel_ref[0, pidx_s] / 100.0
            efac = jnp.where(relv >= 0.95, 0.01 * relv, 0.0)
            proto = p_ref[pl.ds(pidx_s, 1), :]
            err_ref[pl.ds(t, 1), :] = efac * (proto - a_row)

            gatef = jnp.where(dmin > 0.0001, 1.0, 0.0).astype(jnp.float32)

            cheb = jnp.maximum(jnp.abs(lx - bx_v), jnp.abs(ly - by_v))
            c = jnp.where(cheb == 0, LR[0],
                          jnp.where(cheb == 1, LR[1],
                                    jnp.where(cheb == 2, LR[2], 0.0)))
            c = (c * realf * gatef).astype(jnp.float32)
            omc = 1.0 - c

            m_ref[...] = m_ref[...] * omc + g_col * c
            n_new = n * omc * omc + 2.0 * c * omc * m + c * c * a2

            for dx in range(-2, 3):
                ralign = ((idx_s + (dx * NY - 2)) // 8) * 8
                off = (ralign - (idx_s + dx * NY)) + sub_w
                aoff = jnp.abs(off)
                if abs(dx) == 2:
                    ccol = jnp.where(aoff <= 2, LR[2], 0.0)
                elif abs(dx) == 1:
                    ccol = jnp.where(aoff <= 1, LR[1],
                                     jnp.where(aoff == 2, LR[2], 0.0))
                else:
                    ccol = jnp.where(aoff == 0, LR[0],
                                     jnp.where(aoff == 1, LR[1],
                                               jnp.where(aoff == 2, LR[2],
                                                         0.0)))
                ccol = (ccol * gatef).astype(jnp.float32)
                blk = p_ref[pl.ds(ralign, 16), :]
                p_ref[pl.ds(ralign, 16), :] = blk - ccol * (blk - a_row)
            return n_new

        return jax.lax.fori_loop(0, C, step, n, unroll=True)

    jax.lax.fori_loop(0, NCHUNK, chunk, n0_ref[...])


_PALLAS_KWARGS = dict(
    out_shape=jax.ShapeDtypeStruct((BATCH, DIM), jnp.float32),
    in_specs=[
        pl.BlockSpec(memory_space=pltpu.VMEM),
        pl.BlockSpec(memory_space=pltpu.SMEM),
        pl.BlockSpec(memory_space=pltpu.VMEM),
        pl.BlockSpec(memory_space=pltpu.VMEM),
        pl.BlockSpec(memory_space=pltpu.VMEM),
        pl.BlockSpec(memory_space=pltpu.SMEM),
    ],
    out_specs=pl.BlockSpec(memory_space=pltpu.VMEM),
    scratch_shapes=[
        pltpu.VMEM((NPAD, DIM), jnp.float32),
        pltpu.VMEM((C, NPAD), jnp.float32),
        pltpu.VMEM((BATCH, BATCH), jnp.float32),
    ],
)


@jax.jit
def kernel(activations, labels, som_vectors, cell_labels, cell_reliability):
    x = jnp.arange(GRID)
    rows = ((x[:, None] + XP) * NY + x[None, :] + YP).reshape(-1)
    pflat = som_vectors.reshape(GRID * GRID, DIM)
    p0 = jnp.zeros((NPAD, DIM), jnp.float32).at[rows].set(pflat)
    n0 = jnp.full((NPAD,), BIG, jnp.float32).at[rows].set(
        jnp.sum(pflat * pflat, axis=1)).reshape(1, NPAD)
    cl = jnp.full((NPAD,), -1, jnp.int32).at[rows].set(
        cell_labels.reshape(-1)).reshape(1, NPAD)
    rel = jnp.zeros((NPAD,), jnp.float32).at[rows].set(
        cell_reliability.reshape(-1)).reshape(1, NPAD)
    return pl.pallas_call(_som_body, **_PALLAS_KWARGS)(
        activations, labels.reshape(1, BATCH), p0, n0, cl, rel)


# speedup vs baseline: 44.5547x; 1.0115x over previous
import jax
import jax.numpy as jnp
import numpy as np
from jax.experimental import pallas as pl
from jax.experimental.pallas import tpu as pltpu

GRID = 32
DIM = 256
BATCH = 1024
XP = 2
YP = 4
NX = GRID + 2 * XP
NY = GRID + 2 * YP
NPAD = NX * NY + 16
C = 16
NCHUNK = BATCH // C
BIG = np.float32(3.0e38)
LR = [np.float32(0.2), np.float32(0.1), np.float32(0.05)]


def _som_body(a_ref, lab_ref, p0_ref, n0_ref, cl_ref, rel_ref, err_ref,
              p_ref, m_ref, g_ref):
    p_ref[...] = p0_ref[...]
    g_ref[...] = jax.lax.dot_general(
        a_ref[...], a_ref[...], (((1,), (1,)), ((), ())),
        preferred_element_type=jnp.float32,
        precision=jax.lax.Precision.HIGHEST)
    lane = jax.lax.broadcasted_iota(jnp.int32, (1, NPAD), 1)
    lx = lane // NY
    ly = lane - lx * NY
    lane_b = jax.lax.broadcasted_iota(jnp.int32, (1, BATCH), 1)
    sub_c = jax.lax.broadcasted_iota(jnp.int32, (C, 1), 0)
    sub_w = jax.lax.broadcasted_iota(jnp.int32, (16, 1), 0)
    cl_row = cl_ref[...]
    realf = jnp.where(cl_row >= 0, 1.0, 0.0).astype(jnp.float32)

    def chunk(cidx, n):
        t0 = cidx * C
        a_chunk = a_ref[pl.ds(t0, C), :]
        m_ref[...] = jax.lax.dot_general(
            a_chunk, p_ref[...], (((1,), (1,)), ((), ())),
            preferred_element_type=jnp.float32,
            precision=jax.lax.Precision.HIGHEST)

        def step(k, n):
            t = t0 + k
            m = m_ref[pl.ds(k, 1), :]
            gb = g_ref[pl.ds(t0, C), :]
            g_col = jnp.sum(jnp.where(lane_b == t, gb, 0.0),
                            axis=1, keepdims=True)
            a2 = jnp.sum(jnp.where(sub_c == k, g_col, 0.0),
                         axis=0, keepdims=True)
            b = n - 2.0 * m

            bmin = jnp.min(b, axis=1, keepdims=True)
            eqm = jnp.where(b == bmin, lane, NPAD)
            idx_v = jnp.min(eqm, axis=1, keepdims=True)
            bx_v = idx_v // NY
            by_v = idx_v - bx_v * NY
            idx_s = jnp.min(eqm)
            dmin = bmin + a2

            lab = lab_ref[0, t]
            bp = jnp.where(cl_row == lab, b, BIG)
            pbmin = jnp.min(bp, axis=1, keepdims=True)
            pidx_s = jnp.min(jnp.where((bp == pbmin) & (cl_row >= 0),
                                       lane, NPAD))

            a_row = a_ref[pl.ds(t, 1), :]
            relv = rel_ref[0, pidx_s] / 100.0
            efac = jnp.where(relv >= 0.95, 0.01 * relv, 0.0)
            proto = p_ref[pl.ds(pidx_s, 1), :]
            err_ref[pl.ds(t, 1), :] = efac * (proto - a_row)

            gatef = jnp.where(dmin > 0.0001, 1.0, 0.0).astype(jnp.float32)

            cheb = jnp.maximum(jnp.abs(lx - bx_v), jnp.abs(ly - by_v))
            c = jnp.where(cheb == 0, LR[0],
                          jnp.where(cheb == 1, LR[1],
                                    jnp.where(cheb == 2, LR[2], 0.0)))
            c = (c * realf * gatef).astype(jnp.float32)

            m_ref[...] = m_ref[...] + c * (g_col - m_ref[...])
            d = b + a2
            n_new = n + c * (2.0 * (m - n) + c * d)

            for dx in range(-2, 3):
                ralign = ((idx_s + (dx * NY - 2)) // 8) * 8
                off = (ralign - (idx_s + dx * NY)) + sub_w
                aoff = jnp.abs(off)
                if abs(dx) == 2:
                    ccol = jnp.where(aoff <= 2, LR[2], 0.0)
                elif abs(dx) == 1:
                    ccol = jnp.where(aoff <= 1, LR[1],
                                     jnp.where(aoff == 2, LR[2], 0.0))
                else:
                    ccol = jnp.where(aoff == 0, LR[0],
                                     jnp.where(aoff == 1, LR[1],
                                               jnp.where(aoff == 2, LR[2],
                                                         0.0)))
                ccol = (ccol * gatef).astype(jnp.float32)
                blk = p_ref[pl.ds(ralign, 16), :]
                p_ref[pl.ds(ralign, 16), :] = blk - ccol * (blk - a_row)
            return n_new

        return jax.lax.fori_loop(0, C, step, n, unroll=True)

    jax.lax.fori_loop(0, NCHUNK, chunk, n0_ref[...])


_PALLAS_KWARGS = dict(
    out_shape=jax.ShapeDtypeStruct((BATCH, DIM), jnp.float32),
    in_specs=[
        pl.BlockSpec(memory_space=pltpu.VMEM),
        pl.BlockSpec(memory_space=pltpu.SMEM),
        pl.BlockSpec(memory_space=pltpu.VMEM),
        pl.BlockSpec(memory_space=pltpu.VMEM),
        pl.BlockSpec(memory_space=pltpu.VMEM),
        pl.BlockSpec(memory_space=pltpu.SMEM),
    ],
    out_specs=pl.BlockSpec(memory_space=pltpu.VMEM),
    scratch_shapes=[
        pltpu.VMEM((NPAD, DIM), jnp.float32),
        pltpu.VMEM((C, NPAD), jnp.float32),
        pltpu.VMEM((BATCH, BATCH), jnp.float32),
    ],
)


@jax.jit
def kernel(activations, labels, som_vectors, cell_labels, cell_reliability):
    x = jnp.arange(GRID)
    rows = ((x[:, None] + XP) * NY + x[None, :] + YP).reshape(-1)
    pflat = som_vectors.reshape(GRID * GRID, DIM)
    p0 = jnp.zeros((NPAD, DIM), jnp.float32).at[rows].set(pflat)
    n0 = jnp.full((NPAD,), BIG, jnp.float32).at[rows].set(
        jnp.sum(pflat * pflat, axis=1)).reshape(1, NPAD)
    cl = jnp.full((NPAD,), -1, jnp.int32).at[rows].set(
        cell_labels.reshape(-1)).reshape(1, NPAD)
    rel = jnp.zeros((NPAD,), jnp.float32).at[rows].set(
        cell_reliability.reshape(-1)).reshape(1, NPAD)
    return pl.pallas_call(_som_body, **_PALLAS_KWARGS)(
        activations, labels.reshape(1, BATCH), p0, n0, cl, rel)


# speedup vs baseline: 44.9379x; 1.0086x over previous
import jax
import jax.numpy as jnp
import numpy as np
from jax.experimental import pallas as pl
from jax.experimental.pallas import tpu as pltpu

GRID = 32
DIM = 256
BATCH = 1024
XP = 2
YP = 4
NX = GRID + 2 * XP
NY = GRID + 2 * YP
NPAD = NX * NY + 16
C = 16
NCHUNK = BATCH // C
BIG = np.float32(3.0e38)
LR = [np.float32(0.2), np.float32(0.1), np.float32(0.05)]


def _som_body(a_ref, lab_ref, p0_ref, n0_ref, cl_ref, rel_ref, err_ref,
              p_ref, m_ref, g_ref):
    p_ref[...] = p0_ref[...]
    g_ref[...] = jax.lax.dot_general(
        a_ref[...], a_ref[...], (((1,), (1,)), ((), ())),
        preferred_element_type=jnp.float32,
        precision=jax.lax.Precision.HIGHEST)
    lane = jax.lax.broadcasted_iota(jnp.int32, (1, NPAD), 1)
    lx = lane // NY
    ly = lane - lx * NY
    lane_b = jax.lax.broadcasted_iota(jnp.int32, (1, BATCH), 1)
    sub_c = jax.lax.broadcasted_iota(jnp.int32, (C, 1), 0)
    sub_w = jax.lax.broadcasted_iota(jnp.int32, (16, 1), 0)
    cl_row = cl_ref[...]
    realf = jnp.where(cl_row >= 0, 1.0, 0.0).astype(jnp.float32)

    def chunk(cidx, n):
        t0 = cidx * C
        a_chunk = a_ref[pl.ds(t0, C), :]
        m_ref[...] = jax.lax.dot_general(
            a_chunk, p_ref[...], (((1,), (1,)), ((), ())),
            preferred_element_type=jnp.float32,
            precision=jax.lax.Precision.HIGHEST)

        def step(k, n):
            t = t0 + k
            m = m_ref[pl.ds(k, 1), :]
            gb = g_ref[pl.ds(t0, C), :]
            g_col = jnp.sum(jnp.where(lane_b == t, gb, 0.0),
                            axis=1, keepdims=True)
            a2 = jnp.sum(jnp.where(sub_c == k, g_col, 0.0),
                         axis=0, keepdims=True)
            b = n - 2.0 * m

            bmin = jnp.min(b, axis=1, keepdims=True)
            eqm = jnp.where(b == bmin, lane, NPAD)
            idx_v = jnp.min(eqm, axis=1, keepdims=True)
            bx_v = idx_v // NY
            by_v = idx_v - bx_v * NY
            idx_s = jnp.min(eqm)
            dmin = bmin + a2

            lab = lab_ref[0, t]
            bp = jnp.where(cl_row == lab, b, BIG)
            pbmin = jnp.min(bp, axis=1, keepdims=True)
            pidx_s = jnp.min(jnp.where((bp == pbmin) & (cl_row >= 0),
                                       lane, NPAD))

            a_row = a_ref[pl.ds(t, 1), :]
            relv = rel_ref[0, pidx_s] / 100.0
            efac = jnp.where(relv >= 0.95, 0.01 * relv, 0.0)
            proto = p_ref[pl.ds(pidx_s, 1), :]
            err_ref[pl.ds(t, 1), :] = efac * (proto - a_row)

            gatef = jnp.where(dmin > 0.0001, 1.0, 0.0).astype(jnp.float32)

            cheb = jnp.maximum(jnp.abs(lx - bx_v), jnp.abs(ly - by_v))
            c = jnp.where(cheb == 0, LR[0],
                          jnp.where(cheb == 1, LR[1],
                                    jnp.where(cheb == 2, LR[2], 0.0)))
            c = (c * realf * gatef).astype(jnp.float32)

            m_ref[...] = m_ref[...] + c * (g_col - m_ref[...])
            d = b + a2
            n_new = n + 2.0 * (c * (m - n)) + (c * c) * d

            for dx in range(-2, 3):
                ralign = ((idx_s + (dx * NY - 2)) // 8) * 8
                off = (ralign - (idx_s + dx * NY)) + sub_w
                aoff = jnp.abs(off)
                if abs(dx) == 2:
                    ccol = jnp.where(aoff <= 2, LR[2], 0.0)
                elif abs(dx) == 1:
                    ccol = jnp.where(aoff <= 1, LR[1],
                                     jnp.where(aoff == 2, LR[2], 0.0))
                else:
                    ccol = jnp.where(aoff == 0, LR[0],
                                     jnp.where(aoff == 1, LR[1],
                                               jnp.where(aoff == 2, LR[2],
                                                         0.0)))
                ccol = (ccol * gatef).astype(jnp.float32)
                blk = p_ref[pl.ds(ralign, 16), :]
                p_ref[pl.ds(ralign, 16), :] = blk - ccol * (blk - a_row)
            return n_new

        return jax.lax.fori_loop(0, C, step, n, unroll=True)

    jax.lax.fori_loop(0, NCHUNK, chunk, n0_ref[...])


_PALLAS_KWARGS = dict(
    out_shape=jax.ShapeDtypeStruct((BATCH, DIM), jnp.float32),
    in_specs=[
        pl.BlockSpec(memory_space=pltpu.VMEM),
        pl.BlockSpec(memory_space=pltpu.SMEM),
        pl.BlockSpec(memory_space=pltpu.VMEM),
        pl.BlockSpec(memory_space=pltpu.VMEM),
        pl.BlockSpec(memory_space=pltpu.VMEM),
        pl.BlockSpec(memory_space=pltpu.SMEM),
    ],
    out_specs=pl.BlockSpec(memory_space=pltpu.VMEM),
    scratch_shapes=[
        pltpu.VMEM((NPAD, DIM), jnp.float32),
        pltpu.VMEM((C, NPAD), jnp.float32),
        pltpu.VMEM((BATCH, BATCH), jnp.float32),
    ],
)


@jax.jit
def kernel(activations, labels, som_vectors, cell_labels, cell_reliability):
    x = jnp.arange(GRID)
    rows = ((x[:, None] + XP) * NY + x[None, :] + YP).reshape(-1)
    pflat = som_vectors.reshape(GRID * GRID, DIM)
    p0 = jnp.zeros((NPAD, DIM), jnp.float32).at[rows].set(pflat)
    n0 = jnp.full((NPAD,), BIG, jnp.float32).at[rows].set(
        jnp.sum(pflat * pflat, axis=1)).reshape(1, NPAD)
    cl = jnp.full((NPAD,), -1, jnp.int32).at[rows].set(
        cell_labels.reshape(-1)).reshape(1, NPAD)
    rel = jnp.zeros((NPAD,), jnp.float32).at[rows].set(
        cell_reliability.reshape(-1)).reshape(1, NPAD)
    return pl.pallas_call(_som_body, **_PALLAS_KWARGS)(
        activations, labels.reshape(1, BATCH), p0, n0, cl, rel)
